# Initial kernel scaffold; baseline (speedup 1.0000x reference)
import jax
import jax.numpy as jnp
from jax.experimental import pallas as pl


def kernel(Xs, q_mu, q_log_sigma, eps):
    raise NotImplementedError("write your pallas kernel here")


# speedup vs baseline: 1.3580x; 1.3580x over previous
import functools

import jax
import jax.numpy as jnp
from jax.experimental import pallas as pl
from jax.experimental.pallas import tpu as pltpu


def _mixmil_kernel(qmu_ref, qls_ref, eps_ref, x_ref, out_ref, acc_ref, *, n_bags):
    n = pl.program_id(0)
    beta = qmu_ref[...] + jnp.exp(qls_ref[...]) * eps_ref[...]
    q = beta.shape[0] // 2
    beta_u = beta[:q]
    beta_z = beta[q:]
    b = jnp.sqrt(jnp.mean(beta_z * beta_z, axis=0, keepdims=True))
    eta = beta_z / b

    x = x_ref[0]
    dn = (((1,), (0,)), ((), ()))
    u = jax.lax.dot_general(x, beta_u, dn, preferred_element_type=jnp.float32)
    z = jax.lax.dot_general(x, eta, dn, preferred_element_type=jnp.float32)

    m = jnp.max(u, axis=0, keepdims=True)
    e = jnp.exp(u - m)
    denom = jnp.sum(e, axis=0, keepdims=True)
    num = jnp.sum(e * z, axis=0, keepdims=True)
    acc_ref[pl.ds(n, 1), :] = num / denom

    @pl.when(n == n_bags - 1)
    def _finalize():
        xm = acc_ref[...]
        mean = jnp.mean(xm, axis=0, keepdims=True)
        d = xm - mean
        var = jnp.sum(d * d, axis=0, keepdims=True) / (n_bags - 1)
        out_ref[...] = b * d / jnp.sqrt(var)


def kernel(Xs, q_mu, q_log_sigma, eps):
    n_bags, i_inst, q_dim = Xs.shape
    two_q, p_dim, s_dim = eps.shape
    eps2 = eps.reshape(two_q, p_dim * s_dim)

    out = pl.pallas_call(
        functools.partial(_mixmil_kernel, n_bags=n_bags),
        grid=(n_bags,),
        in_specs=[
            pl.BlockSpec((two_q, p_dim), lambda n: (0, 0)),
            pl.BlockSpec((two_q, p_dim), lambda n: (0, 0)),
            pl.BlockSpec((two_q, p_dim * s_dim), lambda n: (0, 0)),
            pl.BlockSpec((1, i_inst, q_dim), lambda n: (n, 0, 0)),
        ],
        out_specs=pl.BlockSpec((n_bags, p_dim * s_dim), lambda n: (0, 0)),
        out_shape=jax.ShapeDtypeStruct((n_bags, p_dim * s_dim), jnp.float32),
        scratch_shapes=[pltpu.VMEM((n_bags, p_dim * s_dim), jnp.float32)],
    )(q_mu, q_log_sigma, eps2, Xs)
    return out.reshape(n_bags, p_dim, s_dim)
